# Initial kernel scaffold; baseline (speedup 1.0000x reference)
import functools

import jax
import jax.numpy as jnp
from jax import lax
from jax.experimental import pallas as pl
from jax.experimental.pallas import tpu as pltpu
from jax.experimental.pallas import tpu_sc as plsc

_LANES = 16
_CHUNK = 128


def _make_sc_kernel(d, e_pad, chunks_per_worker):
  mesh = plsc.VectorSubcoreMesh(core_axis_name="c", subcore_axis_name="s")
  num_cores = mesh.num_cores

  @functools.partial(
      pl.kernel,
      out_type=jax.ShapeDtypeStruct((e_pad,), jnp.float32),
      mesh=mesh,
      scratch_types=[
          pltpu.VMEM((_CHUNK,), jnp.int32),
          pltpu.VMEM((_CHUNK,), jnp.int32),
          pltpu.VMEM((_CHUNK, d), jnp.float32),
          pltpu.VMEM((_CHUNK, d), jnp.float32),
          pltpu.VMEM((_CHUNK,), jnp.float32),
          pltpu.SemaphoreType.DMA,
          pltpu.SemaphoreType.DMA,
      ],
  )
  def k(xu, xb, ui, bi, out, uidx_v, bidx_v, urows, brows, outv, usem, bsem):
    wid = lax.axis_index("s") * num_cores + lax.axis_index("c")
    tile_base = wid * (chunks_per_worker * _CHUNK)

    def edge_body(e, carry):
      acc = urows[e, pl.ds(0, _LANES)] * brows[e, pl.ds(0, _LANES)]
      for j in range(1, d // _LANES):
        acc = acc + (urows[e, pl.ds(j * _LANES, _LANES)] *
                     brows[e, pl.ds(j * _LANES, _LANES)])
      outv[e] = jnp.sum(acc)
      return carry

    def chunk_body(g, carry):
      base = tile_base + g * _CHUNK
      pltpu.sync_copy(ui.at[pl.ds(base, _CHUNK)], uidx_v)
      pltpu.sync_copy(bi.at[pl.ds(base, _CHUNK)], bidx_v)
      cu = pltpu.async_copy(xu.at[uidx_v], urows, usem)
      cb = pltpu.async_copy(xb.at[bidx_v], brows, bsem)
      cu.wait()
      cb.wait()
      lax.fori_loop(0, _CHUNK, edge_body, 0, unroll=4)
      pltpu.sync_copy(outv, out.at[pl.ds(base, _CHUNK)])
      return carry

    lax.fori_loop(0, chunks_per_worker, chunk_body, 0)

  return k


@jax.jit
def kernel(x_user, x_book, edge_label_index):
  d = x_user.shape[1]
  e = edge_label_index.shape[1]

  info = plsc.get_sparse_core_info()
  n_workers = info.num_cores * info.num_subcores
  per_worker = -(-e // (n_workers * _CHUNK))
  e_pad = n_workers * per_worker * _CHUNK

  u_idx = jnp.pad(edge_label_index[0], (0, e_pad - e))
  b_idx = jnp.pad(edge_label_index[1], (0, e_pad - e))

  k = _make_sc_kernel(d, e_pad, per_worker)
  out = k(x_user, x_book, u_idx, b_idx)
  return out[:e]


# speedup vs baseline: 5.2733x; 5.2733x over previous
import functools

import jax
import jax.numpy as jnp
from jax import lax
from jax.experimental import pallas as pl
from jax.experimental.pallas import tpu as pltpu
from jax.experimental.pallas import tpu_sc as plsc

_LANES = 16
_CHUNK = 128


def _make_sc_kernel(d, e_pad, chunks_per_worker):
  mesh = plsc.VectorSubcoreMesh(core_axis_name="c", subcore_axis_name="s")
  num_cores = mesh.num_cores

  @functools.partial(
      pl.kernel,
      out_type=jax.ShapeDtypeStruct((e_pad,), jnp.float32),
      mesh=mesh,
      scratch_types=[
          pltpu.VMEM((_CHUNK,), jnp.int32),
          pltpu.VMEM((_CHUNK,), jnp.int32),
          pltpu.VMEM((_CHUNK, d), jnp.float32),
          pltpu.VMEM((_CHUNK, d), jnp.float32),
          pltpu.VMEM((_CHUNK,), jnp.float32),
          pltpu.SemaphoreType.DMA,
          pltpu.SemaphoreType.DMA,
      ],
      compiler_params=pltpu.CompilerParams(use_tc_tiling_on_sc=False),
  )
  def k(xu, xb, ui, bi, out, uidx_v, bidx_v, urows, brows, outv, usem, bsem):
    wid = lax.axis_index("s") * num_cores + lax.axis_index("c")
    tile_base = wid * (chunks_per_worker * _CHUNK)

    lane_iota = lax.iota(jnp.int32, _LANES)

    shuffle_dnums = lax.GatherDimensionNumbers(
        offset_dims=(), collapsed_slice_dims=(0,), start_index_map=(0,))

    def _shuffle(v, perm):
      return lax.gather(
          v, perm[:, None], shuffle_dnums, (1,),
          indices_are_sorted=False, unique_indices=False,
          mode=lax.GatherScatterMode.PROMISE_IN_BOUNDS)

    def group_body(grp, carry):
      vecs = []
      for i in range(_LANES):
        e = grp * _LANES + i
        acc = urows[e, pl.ds(0, _LANES)] * brows[e, pl.ds(0, _LANES)]
        for j in range(1, d // _LANES):
          acc = acc + (urows[e, pl.ds(j * _LANES, _LANES)] *
                       brows[e, pl.ds(j * _LANES, _LANES)])
        vecs.append(acc)
      s = _LANES // 2
      while s >= 1:
        m = (lane_iota & s) == 0
        perm = lane_iota ^ s
        vecs = [
            jnp.where(m, a, b) + _shuffle(jnp.where(m, b, a), perm)
            for a, b in zip(vecs[:s], vecs[s:])
        ]
        s //= 2
      outv[pl.ds(grp * _LANES, _LANES)] = vecs[0]
      return carry

    def chunk_body(g, carry):
      base = tile_base + g * _CHUNK
      pltpu.sync_copy(ui.at[pl.ds(base, _CHUNK)], uidx_v)
      pltpu.sync_copy(bi.at[pl.ds(base, _CHUNK)], bidx_v)
      cu = pltpu.async_copy(xu.at[uidx_v], urows, usem)
      cb = pltpu.async_copy(xb.at[bidx_v], brows, bsem)
      cu.wait()
      cb.wait()
      lax.fori_loop(0, _CHUNK // _LANES, group_body, 0)
      pltpu.sync_copy(outv, out.at[pl.ds(base, _CHUNK)])
      return carry

    lax.fori_loop(0, chunks_per_worker, chunk_body, 0)

  return k


@jax.jit
def kernel(x_user, x_book, edge_label_index):
  d = x_user.shape[1]
  e = edge_label_index.shape[1]

  info = plsc.get_sparse_core_info()
  n_workers = info.num_cores * info.num_subcores
  per_worker = -(-e // (n_workers * _CHUNK))
  e_pad = n_workers * per_worker * _CHUNK

  u_idx = jnp.pad(edge_label_index[0], (0, e_pad - e))
  b_idx = jnp.pad(edge_label_index[1], (0, e_pad - e))

  k = _make_sc_kernel(d, e_pad, per_worker)
  out = k(x_user, x_book, u_idx, b_idx)
  return out[:e]
